# Initial kernel scaffold; baseline (speedup 1.0000x reference)
import jax
import jax.numpy as jnp
from jax.experimental import pallas as pl


def kernel(target, context, emb_table, ctx_table):
    raise NotImplementedError("write your pallas kernel here")


# speedup vs baseline: 1.7346x; 1.7346x over previous
import functools

import jax
import jax.numpy as jnp
from jax import lax
from jax.experimental import pallas as pl
from jax.experimental.pallas import tpu as pltpu
from jax.experimental.pallas import tpu_sc as plsc

VOCAB = 1000000
ENC = 64
BATCH = 16384
NCTX = 50

NC = 2
NS = 16
NW = NC * NS
BPW = BATCH // NW
CB = 16
NCHUNK = BPW // CB
NK = ENC // 16


def _sc_body(target_hbm, context_hbm, emb_hbm, ctxt_hbm, out_hbm,
             tgt_idx_v, ctx_idx_v, tgt_rows_v, ctx_rows_v, out_v, gsem):
    cid = lax.axis_index("c")
    sid = lax.axis_index("s")
    wid = sid * NC + cid
    base = wid * BPW

    lanes = lax.iota(jnp.int32, 16)

    pltpu.sync_copy(target_hbm.at[pl.ds(base, BPW)], tgt_idx_v)

    def chunk(g, carry):
        row0 = base + g * CB
        pltpu.sync_copy(context_hbm.at[pl.ds(row0, CB)], ctx_idx_v)

        tcp = pltpu.async_copy(
            emb_hbm.at[tgt_idx_v.at[pl.ds(g * CB, CB)]], tgt_rows_v, gsem)
        ccps = [
            pltpu.async_copy(
                ctxt_hbm.at[ctx_idx_v.at[b]], ctx_rows_v.at[b], gsem)
            for b in range(CB)
        ]
        tcp.wait()
        for cp in ccps:
            cp.wait()

        def bbody(b, _):
            t = [tgt_rows_v[b, pl.ds(k * 16, 16)] for k in range(NK)]
            for cg in range(4):
                n = min(16, NCTX - cg * 16)
                vec = jnp.zeros((16,), jnp.float32)
                for j in range(n):
                    c = cg * 16 + j
                    m = t[0] * ctx_rows_v[b, c, pl.ds(0, 16)]
                    for k in range(1, NK):
                        m = m + t[k] * ctx_rows_v[b, c, pl.ds(k * 16, 16)]
                    r = jnp.sum(m)
                    vec = jnp.where(lanes == j, r, vec)
                out_v[b, pl.ds(cg * 16, 16)] = vec
            return 0

        lax.fori_loop(0, CB, bbody, 0)
        pltpu.sync_copy(out_v, out_hbm.at[pl.ds(row0, CB)])
        return 0

    lax.fori_loop(0, NCHUNK, chunk, 0)


@jax.jit
def kernel(target, context, emb_table, ctx_table):
    mesh = plsc.VectorSubcoreMesh(core_axis_name="c", subcore_axis_name="s")
    run = functools.partial(
        pl.kernel,
        mesh=mesh,
        out_type=jax.ShapeDtypeStruct((BATCH, 64), jnp.float32),
        compiler_params=pltpu.CompilerParams(
            needs_layout_passes=False, use_tc_tiling_on_sc=False),
        scratch_types=[
            pltpu.VMEM((BPW,), jnp.int32),
            pltpu.VMEM((CB, NCTX), jnp.int32),
            pltpu.VMEM((CB, ENC), jnp.float32),
            pltpu.VMEM((CB, NCTX, ENC), jnp.float32),
            pltpu.VMEM((CB, 64), jnp.float32),
            pltpu.SemaphoreType.DMA,
        ],
    )(_sc_body)
    out = run(target.astype(jnp.int32), context.astype(jnp.int32),
              emb_table, ctx_table)
    return out[:, :NCTX]


# speedup vs baseline: 1.7470x; 1.0072x over previous
import functools

import jax
import jax.numpy as jnp
from jax import lax
from jax.experimental import pallas as pl
from jax.experimental.pallas import tpu as pltpu
from jax.experimental.pallas import tpu_sc as plsc

VOCAB = 1000000
ENC = 64
BATCH = 16384
NCTX = 50

NC = 2
NS = 16
NW = NC * NS
BPW = BATCH // NW
CB = 16
NCHUNK = BPW // CB
NK = ENC // 16
NPC = CB * NCTX
IMAX = VOCAB - 1


def _lk_body(target_hbm, ctxflat_hbm, emb_hbm, ctxt_hbm, out_hbm,
             tgt_idx_v, ctx_idx_v, tgt_rows_v, ctx_rows_v, out_v, gsem):
    cid = lax.axis_index("c")
    sid = lax.axis_index("s")
    wid = sid * NC + cid
    base = wid * BPW

    lanes = lax.iota(jnp.int32, 16)

    pltpu.sync_copy(target_hbm.at[pl.ds(base, BPW)], tgt_idx_v)

    def chunk(g, carry):
        row0 = base + g * CB
        pltpu.sync_copy(ctxflat_hbm.at[pl.ds(row0 * NCTX, NPC)], ctx_idx_v)

        tiv = tgt_idx_v[pl.ds(g * CB, CB)]

        tcp = pltpu.async_copy(emb_hbm.at[tiv], tgt_rows_v, gsem)
        ccps = [
            pltpu.async_copy(
                ctxt_hbm.at[ctx_idx_v.at[pl.ds(j * 32, 32)]],
                ctx_rows_v.at[pl.ds(j * 32, 32)], gsem)
            for j in range(NPC // 32)
        ]
        tcp.wait()
        for cp in ccps:
            cp.wait()

        def bbody(b, _):
            t = [tgt_rows_v[b, pl.ds(k * 16, 16)] for k in range(NK)]
            for cg in range(4):
                n = min(16, NCTX - cg * 16)
                vec = jnp.zeros((16,), jnp.float32)
                for j in range(n):
                    p = b * NCTX + cg * 16 + j
                    m = t[0] * ctx_rows_v[p, pl.ds(0, 16)]
                    for k in range(1, NK):
                        m = m + t[k] * ctx_rows_v[p, pl.ds(k * 16, 16)]
                    r = jnp.sum(m)
                    vec = jnp.where(lanes == j, r, vec)
                out_v[b, pl.ds(cg * 16, 16)] = vec
            return 0

        lax.fori_loop(0, CB, bbody, 0)
        pltpu.sync_copy(out_v, out_hbm.at[pl.ds(row0, CB)])
        return 0

    lax.fori_loop(0, NCHUNK, chunk, 0)


@jax.jit
def kernel(target, context, emb_table, ctx_table):
    mesh = plsc.VectorSubcoreMesh(core_axis_name="c", subcore_axis_name="s")
    run = functools.partial(
        pl.kernel,
        mesh=mesh,
        out_type=jax.ShapeDtypeStruct((BATCH, 128), jnp.float32),
        compiler_params=pltpu.CompilerParams(needs_layout_passes=False),
        scratch_types=[
            pltpu.VMEM((BPW,), jnp.int32),
            pltpu.VMEM((NPC,), jnp.int32),
            pltpu.VMEM((CB, 128), jnp.float32),
            pltpu.VMEM((NPC, 128), jnp.float32),
            pltpu.VMEM((CB, 128), jnp.float32),
            pltpu.SemaphoreType.DMA,
        ],
    )(_lk_body)
    emb3 = jnp.pad(emb_table, ((0, 0), (0, 128 - ENC)))
    ctx3 = jnp.pad(ctx_table, ((0, 0), (0, 128 - ENC)))
    ctxflat = context.astype(jnp.int32).reshape(BATCH * NCTX)
    out = run(target.astype(jnp.int32), ctxflat, emb3, ctx3)
    return out[:, :NCTX]


# speedup vs baseline: 1.8799x; 1.0760x over previous
import functools

import jax
import jax.numpy as jnp
from jax import lax
from jax.experimental import pallas as pl
from jax.experimental.pallas import tpu as pltpu
from jax.experimental.pallas import tpu_sc as plsc

VOCAB = 1000000
ENC = 64
BATCH = 16384
NCTX = 50

NC = 2
NS = 16
NW = NC * NS
BPW = BATCH // NW
CB = 8
NCHUNK = BPW // CB
NK = ENC // 16
NPC = CB * NCTX
GSZ = 80
NGB = NPC // GSZ


def _lk_body(target_hbm, ctxflat_hbm, emb_hbm, ctxt_hbm, out_hbm,
             tgt_idx_v, ci0_v, ci1_v, tr0_v, tr1_v, cr0_v, cr1_v,
             out_v, sem0, sem1):
    cid = lax.axis_index("c")
    sid = lax.axis_index("s")
    wid = sid * NC + cid
    base = wid * BPW

    lanes = lax.iota(jnp.int32, 16)
    cis = [ci0_v, ci1_v]
    trs = [tr0_v, tr1_v]
    crs = [cr0_v, cr1_v]
    sems = [sem0, sem1]

    pltpu.sync_copy(target_hbm.at[pl.ds(base, BPW)], tgt_idx_v)

    def stage(g, sub):
        row0 = base + g * CB
        pltpu.sync_copy(ctxflat_hbm.at[pl.ds(row0 * NCTX, NPC)], cis[sub])
        pltpu.make_async_copy(
            emb_hbm.at[tgt_idx_v.at[pl.ds(g * CB, CB)]],
            trs[sub], sems[sub]).start()
        for j in range(NGB):
            pltpu.make_async_copy(
                ctxt_hbm.at[cis[sub].at[pl.ds(j * GSZ, GSZ)]],
                crs[sub].at[pl.ds(j * GSZ, GSZ)], sems[sub]).start()

    def drain(sub):
        pltpu.make_async_copy(emb_hbm.at[tgt_idx_v.at[pl.ds(0, CB)]],
                              trs[sub], sems[sub]).wait()
        for j in range(NGB):
            pltpu.make_async_copy(
                ctxt_hbm.at[cis[sub].at[pl.ds(j * GSZ, GSZ)]],
                crs[sub].at[pl.ds(j * GSZ, GSZ)], sems[sub]).wait()

    def compute(g, sub):
        row0 = base + g * CB

        def bbody(b, _):
            t = [trs[sub][b, pl.ds(k * 16, 16)] for k in range(NK)]
            for cg in range(4):
                n = min(16, NCTX - cg * 16)
                vec = jnp.zeros((16,), jnp.float32)
                for j in range(n):
                    p = b * NCTX + cg * 16 + j
                    m = t[0] * crs[sub][p, pl.ds(0, 16)]
                    for k in range(1, NK):
                        m = m + t[k] * crs[sub][p, pl.ds(k * 16, 16)]
                    r = jnp.sum(m)
                    vec = jnp.where(lanes == j, r, vec)
                out_v[b, pl.ds(cg * 16, 16)] = vec
            return 0

        lax.fori_loop(0, CB, bbody, 0)
        pltpu.sync_copy(out_v, out_hbm.at[pl.ds(row0, CB)])

    stage(0, 0)
    stage(1, 1)

    def chunk2(h, carry):
        for sub in range(2):
            g = 2 * h + sub
            drain(sub)
            compute(g, sub)
            stage(jnp.minimum(g + 2, NCHUNK - 1), sub)
        return 0

    lax.fori_loop(0, NCHUNK // 2, chunk2, 0)
    drain(0)
    drain(1)


@jax.jit
def kernel(target, context, emb_table, ctx_table):
    mesh = plsc.VectorSubcoreMesh(core_axis_name="c", subcore_axis_name="s")
    run = functools.partial(
        pl.kernel,
        mesh=mesh,
        out_type=jax.ShapeDtypeStruct((BATCH, 128), jnp.float32),
        compiler_params=pltpu.CompilerParams(needs_layout_passes=False),
        scratch_types=[
            pltpu.VMEM((BPW,), jnp.int32),
            pltpu.VMEM((NPC,), jnp.int32),
            pltpu.VMEM((NPC,), jnp.int32),
            pltpu.VMEM((CB, 128), jnp.float32),
            pltpu.VMEM((CB, 128), jnp.float32),
            pltpu.VMEM((NPC, 128), jnp.float32),
            pltpu.VMEM((NPC, 128), jnp.float32),
            pltpu.VMEM((CB, 128), jnp.float32),
            pltpu.SemaphoreType.DMA,
            pltpu.SemaphoreType.DMA,
        ],
    )(_lk_body)
    emb3 = jnp.pad(emb_table, ((0, 0), (0, 128 - ENC)))
    ctx3 = jnp.pad(ctx_table, ((0, 0), (0, 128 - ENC)))
    ctxflat = context.astype(jnp.int32).reshape(BATCH * NCTX)
    out = run(target.astype(jnp.int32), ctxflat, emb3, ctx3)
    return out[:, :NCTX]
